# Initial kernel scaffold; baseline (speedup 1.0000x reference)
import functools

import jax
import jax.numpy as jnp
from jax import lax
from jax.experimental import pallas as pl
from jax.experimental.pallas import tpu as pltpu
from jax.experimental.pallas import tpu_sc as plsc

N_USERS = 6000
N_MOVIES = 4000
N = N_USERS + N_MOVIES
N_PAD = 10240
E = 320000
D = 64
DW = 16
BN_EPS = 1e-5

NC = 2
NS = 16
NW = NC * NS
EPW = E // NW
B = 80
NBLK = EPW // B
ROWS_PER_TILE = N_PAD // NS


def _vector_mesh():
    return plsc.VectorSubcoreMesh(core_axis_name="c", subcore_axis_name="s",
                                  num_cores=NC, num_subcores=NS)


def _sc_degree(dst, zeros16, ones_blk):
    @functools.partial(
        pl.kernel,
        out_type=jax.ShapeDtypeStruct((NC, N_PAD, DW), jnp.float32),
        mesh=_vector_mesh(),
        scratch_types=[
            pltpu.VMEM((B,), jnp.int32),
            pltpu.VMEM((B, DW), jnp.float32),
            pltpu.VMEM_SHARED((N_PAD, DW), jnp.float32),
        ],
    )
    def deg_kernel(dst_hbm, z_hbm, ones_hbm, out_hbm, idx_v, ones_v, acc_sh):
        c = lax.axis_index("c")
        s = lax.axis_index("s")
        wid = s * NC + c
        row0 = s * ROWS_PER_TILE
        pltpu.sync_copy(z_hbm.at[pl.ds(row0, ROWS_PER_TILE)],
                        acc_sh.at[pl.ds(row0, ROWS_PER_TILE)])
        pltpu.sync_copy(ones_hbm, ones_v)
        plsc.subcore_barrier()

        @pl.loop(0, NBLK)
        def _(blk):
            base = wid * EPW + blk * B
            pltpu.sync_copy(dst_hbm.at[pl.ds(base, B)], idx_v)
            pltpu.sync_copy(ones_v, acc_sh.at[idx_v], add=True)

        plsc.subcore_barrier()
        pltpu.sync_copy(acc_sh.at[pl.ds(row0, ROWS_PER_TILE)],
                        out_hbm.at[c, pl.ds(row0, ROWS_PER_TILE)])

    return deg_kernel(dst, zeros16, ones_blk)


def _sc_aggregate(h, src, dst, zeros64):
    @functools.partial(
        pl.kernel,
        out_type=jax.ShapeDtypeStruct((NC, N_PAD, D), jnp.float32),
        mesh=_vector_mesh(),
        scratch_types=[
            pltpu.VMEM((B,), jnp.int32),
            pltpu.VMEM((B,), jnp.int32),
            pltpu.VMEM((B, D), jnp.float32),
            pltpu.VMEM_SHARED((N_PAD, D), jnp.float32),
            pltpu.SemaphoreType.DMA,
        ],
    )
    def agg_kernel(h_hbm, src_hbm, dst_hbm, z_hbm, out_hbm,
                   sidx_v, didx_v, rows_v, acc_sh, sem):
        c = lax.axis_index("c")
        s = lax.axis_index("s")
        wid = s * NC + c
        row0 = s * ROWS_PER_TILE
        pltpu.sync_copy(z_hbm.at[pl.ds(row0, ROWS_PER_TILE)],
                        acc_sh.at[pl.ds(row0, ROWS_PER_TILE)])
        plsc.subcore_barrier()

        @pl.loop(0, NBLK)
        def _(blk):
            base = wid * EPW + blk * B
            pltpu.sync_copy(src_hbm.at[pl.ds(base, B)], sidx_v)
            pltpu.sync_copy(dst_hbm.at[pl.ds(base, B)], didx_v)
            pltpu.async_copy(h_hbm.at[sidx_v], rows_v, sem).wait()
            pltpu.sync_copy(rows_v, acc_sh.at[didx_v], add=True)

        plsc.subcore_barrier()
        pltpu.sync_copy(acc_sh.at[pl.ds(row0, ROWS_PER_TILE)],
                        out_hbm.at[c, pl.ds(row0, ROWS_PER_TILE)])

    return agg_kernel(h, src, dst, zeros64)


def _tc_prologue(x, y, f1w, f1b, f2w, f2b, c1w, degp):
    def body(x_ref, y_ref, f1w_ref, f1b_ref, f2w_ref, f2b_ref, c1w_ref,
             degp_ref, h1s_ref, dinv_ref):
        zx = jnp.dot(x_ref[...], f1w_ref[...],
                     preferred_element_type=jnp.float32) + f1b_ref[...]
        zy = jnp.dot(y_ref[...], f2w_ref[...],
                     preferred_element_type=jnp.float32) + f2b_ref[...]
        z = jnp.concatenate([zx, zy], axis=0)
        h1 = jnp.dot(z, c1w_ref[...], preferred_element_type=jnp.float32)
        deg = degp_ref[0, :N, :1] + degp_ref[1, :N, :1] + 1.0
        dinv = lax.rsqrt(deg)
        h1s_ref[...] = h1 * dinv
        dinv_ref[...] = dinv

    return pl.pallas_call(
        body,
        out_shape=(jax.ShapeDtypeStruct((N, D), jnp.float32),
                   jax.ShapeDtypeStruct((N, 1), jnp.float32)),
    )(x, y, f1w, f1b, f2w, f2b, c1w, degp)


def _tc_mid(p1, h1s, dinv, c1b, gamma, beta, c2w):
    def body(p_ref, h1s_ref, dinv_ref, c1b_ref, g_ref, b_ref, c2w_ref,
             h2s_ref):
        dinv = dinv_ref[...]
        t = (p_ref[0, :N] + p_ref[1, :N] + h1s_ref[...]) * dinv + c1b_ref[...]
        mean = jnp.mean(t, axis=0, keepdims=True)
        var = jnp.mean((t - mean) ** 2, axis=0, keepdims=True)
        r = (t - mean) * lax.rsqrt(var + BN_EPS) * g_ref[...] + b_ref[...]
        r = jnp.maximum(r, 0.0)
        h2 = jnp.dot(r, c2w_ref[...], preferred_element_type=jnp.float32)
        h2s_ref[...] = h2 * dinv

    return pl.pallas_call(
        body,
        out_shape=jax.ShapeDtypeStruct((N, D), jnp.float32),
    )(p1, h1s, dinv, c1b, gamma, beta, c2w)


def _tc_epilogue(p2, h2s, dinv, c2b):
    def body(p_ref, h2s_ref, dinv_ref, c2b_ref, out_ref):
        out_ref[...] = ((p_ref[0, :N] + p_ref[1, :N] + h2s_ref[...])
                        * dinv_ref[...] + c2b_ref[...])

    return pl.pallas_call(
        body,
        out_shape=jax.ShapeDtypeStruct((N, D), jnp.float32),
    )(p2, h2s, dinv, c2b)


def kernel(x, y, edge_index, fc1_W, fc1_b, fc2_W, fc2_b,
           conv1_W, conv1_b, conv2_W, conv2_b, bn_gamma, bn_beta):
    src = edge_index[0]
    dst = edge_index[1]
    zeros16 = jnp.zeros((N_PAD, DW), jnp.float32)
    zeros64 = jnp.zeros((N_PAD, D), jnp.float32)
    ones_blk = jnp.ones((B, DW), jnp.float32)

    degp = _sc_degree(dst, zeros16, ones_blk)
    h1s, dinv = _tc_prologue(x, y, fc1_W, fc1_b.reshape(1, -1),
                             fc2_W, fc2_b.reshape(1, -1), conv1_W, degp)
    p1 = _sc_aggregate(h1s, src, dst, zeros64)
    h2s = _tc_mid(p1, h1s, dinv, conv1_b.reshape(1, -1),
                  bn_gamma.reshape(1, -1), bn_beta.reshape(1, -1), conv2_W)
    p2 = _sc_aggregate(h2s, src, dst, zeros64)
    out = _tc_epilogue(p2, h2s, dinv, conv2_b.reshape(1, -1))
    return out


# speedup vs baseline: 14.3942x; 14.3942x over previous
import functools

import jax
import jax.numpy as jnp
from jax import lax
from jax.experimental import pallas as pl
from jax.experimental.pallas import tpu as pltpu
from jax.experimental.pallas import tpu_sc as plsc

N_USERS = 6000
N_MOVIES = 4000
N = N_USERS + N_MOVIES
N_PAD = 10240
E = 320000
D = 64
DW = 16
BN_EPS = 1e-5

NC = 2
NS = 16
NW = NC * NS
EPW = E // NW
B = 80
NBLK = EPW // B
ROWS_PER_TILE = N_PAD // NS


def _vector_mesh():
    return plsc.VectorSubcoreMesh(core_axis_name="c", subcore_axis_name="s",
                                  num_cores=NC, num_subcores=NS)


_SC_PARAMS = pltpu.CompilerParams(use_tc_tiling_on_sc=False)


def _sc_degree(dst, zeros16, ones_blk):
    @functools.partial(
        pl.kernel,
        out_type=jax.ShapeDtypeStruct((NC, N_PAD, DW), jnp.float32),
        mesh=_vector_mesh(),
        compiler_params=_SC_PARAMS,
        scratch_types=[
            pltpu.VMEM((B,), jnp.int32),
            pltpu.VMEM((B, DW), jnp.float32),
            pltpu.VMEM_SHARED((N_PAD, DW), jnp.float32),
        ],
    )
    def deg_kernel(dst_hbm, z_hbm, ones_hbm, out_hbm, idx_v, ones_v, acc_sh):
        c = lax.axis_index("c")
        s = lax.axis_index("s")
        wid = s * NC + c
        row0 = s * ROWS_PER_TILE
        pltpu.sync_copy(z_hbm.at[pl.ds(row0, ROWS_PER_TILE)],
                        acc_sh.at[pl.ds(row0, ROWS_PER_TILE)])
        pltpu.sync_copy(ones_hbm, ones_v)
        plsc.subcore_barrier()

        @pl.loop(0, NBLK)
        def _(blk):
            base = wid * EPW + blk * B
            pltpu.sync_copy(dst_hbm.at[pl.ds(base, B)], idx_v)
            pltpu.sync_copy(ones_v, acc_sh.at[idx_v], add=True)

        plsc.subcore_barrier()
        pltpu.sync_copy(acc_sh.at[pl.ds(row0, ROWS_PER_TILE)],
                        out_hbm.at[c, pl.ds(row0, ROWS_PER_TILE)])

    return deg_kernel(dst, zeros16, ones_blk)


def _sc_aggregate(h, src, dst, zeros64):
    @functools.partial(
        pl.kernel,
        out_type=jax.ShapeDtypeStruct((NC, N_PAD, D), jnp.float32),
        mesh=_vector_mesh(),
        compiler_params=_SC_PARAMS,
        scratch_types=[
            pltpu.VMEM((B,), jnp.int32),
            pltpu.VMEM((B,), jnp.int32),
            pltpu.VMEM((B, D), jnp.float32),
            pltpu.VMEM_SHARED((N_PAD, D), jnp.float32),
            pltpu.SemaphoreType.DMA,
        ],
    )
    def agg_kernel(h_hbm, src_hbm, dst_hbm, z_hbm, out_hbm,
                   sidx_v, didx_v, rows_v, acc_sh, sem):
        c = lax.axis_index("c")
        s = lax.axis_index("s")
        wid = s * NC + c
        row0 = s * ROWS_PER_TILE
        pltpu.sync_copy(z_hbm.at[pl.ds(row0, ROWS_PER_TILE)],
                        acc_sh.at[pl.ds(row0, ROWS_PER_TILE)])
        plsc.subcore_barrier()

        @pl.loop(0, NBLK)
        def _(blk):
            base = wid * EPW + blk * B
            pltpu.sync_copy(src_hbm.at[pl.ds(base, B)], sidx_v)
            pltpu.sync_copy(dst_hbm.at[pl.ds(base, B)], didx_v)
            pltpu.async_copy(h_hbm.at[sidx_v], rows_v, sem).wait()
            pltpu.sync_copy(rows_v, acc_sh.at[didx_v], add=True)

        plsc.subcore_barrier()
        pltpu.sync_copy(acc_sh.at[pl.ds(row0, ROWS_PER_TILE)],
                        out_hbm.at[c, pl.ds(row0, ROWS_PER_TILE)])

    return agg_kernel(h, src, dst, zeros64)


def _tc_prologue(x, y, f1w, f1b, f2w, f2b, c1w, degp):
    def body(x_ref, y_ref, f1w_ref, f1b_ref, f2w_ref, f2b_ref, c1w_ref,
             degp_ref, h1s_ref, dinv_ref):
        zx = jnp.dot(x_ref[...], f1w_ref[...],
                     preferred_element_type=jnp.float32) + f1b_ref[...]
        zy = jnp.dot(y_ref[...], f2w_ref[...],
                     preferred_element_type=jnp.float32) + f2b_ref[...]
        z = jnp.concatenate([zx, zy], axis=0)
        h1 = jnp.dot(z, c1w_ref[...], preferred_element_type=jnp.float32)
        deg = degp_ref[0, :N, :1] + degp_ref[1, :N, :1] + 1.0
        dinv = lax.rsqrt(deg)
        h1s_ref[...] = h1 * dinv
        dinv_ref[...] = dinv

    return pl.pallas_call(
        body,
        out_shape=(jax.ShapeDtypeStruct((N, D), jnp.float32),
                   jax.ShapeDtypeStruct((N, 1), jnp.float32)),
    )(x, y, f1w, f1b, f2w, f2b, c1w, degp)


def _tc_mid(p1, h1s, dinv, c1b, gamma, beta, c2w):
    def body(p_ref, h1s_ref, dinv_ref, c1b_ref, g_ref, b_ref, c2w_ref,
             h2s_ref):
        dinv = dinv_ref[...]
        t = (p_ref[0, :N] + p_ref[1, :N] + h1s_ref[...]) * dinv + c1b_ref[...]
        mean = jnp.mean(t, axis=0, keepdims=True)
        var = jnp.mean((t - mean) ** 2, axis=0, keepdims=True)
        r = (t - mean) * lax.rsqrt(var + BN_EPS) * g_ref[...] + b_ref[...]
        r = jnp.maximum(r, 0.0)
        h2 = jnp.dot(r, c2w_ref[...], preferred_element_type=jnp.float32)
        h2s_ref[...] = h2 * dinv

    return pl.pallas_call(
        body,
        out_shape=jax.ShapeDtypeStruct((N, D), jnp.float32),
    )(p1, h1s, dinv, c1b, gamma, beta, c2w)


def _tc_epilogue(p2, h2s, dinv, c2b):
    def body(p_ref, h2s_ref, dinv_ref, c2b_ref, out_ref):
        out_ref[...] = ((p_ref[0, :N] + p_ref[1, :N] + h2s_ref[...])
                        * dinv_ref[...] + c2b_ref[...])

    return pl.pallas_call(
        body,
        out_shape=jax.ShapeDtypeStruct((N, D), jnp.float32),
    )(p2, h2s, dinv, c2b)


def kernel(x, y, edge_index, fc1_W, fc1_b, fc2_W, fc2_b,
           conv1_W, conv1_b, conv2_W, conv2_b, bn_gamma, bn_beta):
    src = edge_index[0]
    dst = edge_index[1]
    zeros16 = jnp.zeros((N_PAD, DW), jnp.float32)
    zeros64 = jnp.zeros((N_PAD, D), jnp.float32)
    ones_blk = jnp.ones((B, DW), jnp.float32)

    degp = _sc_degree(dst, zeros16, ones_blk)
    h1s, dinv = _tc_prologue(x, y, fc1_W, fc1_b.reshape(1, -1),
                             fc2_W, fc2_b.reshape(1, -1), conv1_W, degp)
    p1 = _sc_aggregate(h1s, src, dst, zeros64)
    h2s = _tc_mid(p1, h1s, dinv, conv1_b.reshape(1, -1),
                  bn_gamma.reshape(1, -1), bn_beta.reshape(1, -1), conv2_W)
    p2 = _sc_aggregate(h2s, src, dst, zeros64)
    out = _tc_epilogue(p2, h2s, dinv, conv2_b.reshape(1, -1))
    return out


# speedup vs baseline: 35.4575x; 2.4633x over previous
import functools

import jax
import jax.numpy as jnp
from jax import lax
from jax.experimental import pallas as pl
from jax.experimental.pallas import tpu as pltpu
from jax.experimental.pallas import tpu_sc as plsc

N_USERS = 6000
N_MOVIES = 4000
N = N_USERS + N_MOVIES
N_PAD = 10240
E = 320000
D = 64
DW = 16
BN_EPS = 1e-5

NC = 2
NS = 16
NW = NC * NS
EPW = E // NW
B = 125
NBLK = EPW // B
K = 8
ITERS = NBLK // K
ROWS_PER_TILE = N_PAD // NS


def _vector_mesh():
    return plsc.VectorSubcoreMesh(core_axis_name="c", subcore_axis_name="s",
                                  num_cores=NC, num_subcores=NS)


_SC_PARAMS = pltpu.CompilerParams(use_tc_tiling_on_sc=False)


def _sc_degree(dst, zeros16, ones_blk):
    @functools.partial(
        pl.kernel,
        out_type=jax.ShapeDtypeStruct((NC, N_PAD, DW), jnp.float32),
        mesh=_vector_mesh(),
        compiler_params=_SC_PARAMS,
        scratch_types=[
            pltpu.VMEM((NBLK, B), jnp.int32),
            pltpu.VMEM((B, DW), jnp.float32),
            pltpu.VMEM_SHARED((N_PAD, DW), jnp.float32),
            pltpu.SemaphoreType.DMA,
        ],
    )
    def deg_kernel(dst_hbm, z_hbm, ones_hbm, out_hbm, didx_v, ones_v, acc_sh,
                   ssem):
        c = lax.axis_index("c")
        s = lax.axis_index("s")
        wid = s * NC + c
        row0 = s * ROWS_PER_TILE
        pltpu.sync_copy(z_hbm.at[pl.ds(row0, ROWS_PER_TILE)],
                        acc_sh.at[pl.ds(row0, ROWS_PER_TILE)])
        pltpu.sync_copy(dst_hbm.at[wid], didx_v)
        pltpu.sync_copy(ones_hbm, ones_v)
        plsc.subcore_barrier()

        @pl.loop(0, ITERS)
        def _(it):
            blk0 = it * K
            copies = [
                pltpu.async_copy(ones_v, acc_sh.at[didx_v.at[blk0 + j]],
                                 ssem, add=True)
                for j in range(K)
            ]
            for d in copies:
                d.wait()

        plsc.subcore_barrier()
        pltpu.sync_copy(acc_sh.at[pl.ds(row0, ROWS_PER_TILE)],
                        out_hbm.at[c, pl.ds(row0, ROWS_PER_TILE)])

    return deg_kernel(dst, zeros16, ones_blk)


def _sc_aggregate(h, src, dst, zeros64):
    @functools.partial(
        pl.kernel,
        out_type=jax.ShapeDtypeStruct((NC, N_PAD, D), jnp.float32),
        mesh=_vector_mesh(),
        compiler_params=_SC_PARAMS,
        scratch_types=[
            pltpu.VMEM((NBLK, B), jnp.int32),
            pltpu.VMEM((NBLK, B), jnp.int32),
            pltpu.VMEM((K, B, D), jnp.float32),
            pltpu.VMEM_SHARED((N_PAD, D), jnp.float32),
            pltpu.SemaphoreType.DMA,
            pltpu.SemaphoreType.DMA,
        ],
    )
    def agg_kernel(h_hbm, src_hbm, dst_hbm, z_hbm, out_hbm,
                   sidx_v, didx_v, rows_v, acc_sh, gsem, ssem):
        c = lax.axis_index("c")
        s = lax.axis_index("s")
        wid = s * NC + c
        row0 = s * ROWS_PER_TILE
        pltpu.sync_copy(z_hbm.at[pl.ds(row0, ROWS_PER_TILE)],
                        acc_sh.at[pl.ds(row0, ROWS_PER_TILE)])
        pltpu.sync_copy(src_hbm.at[wid], sidx_v)
        pltpu.sync_copy(dst_hbm.at[wid], didx_v)
        plsc.subcore_barrier()

        @pl.loop(0, ITERS)
        def _(it):
            blk0 = it * K
            gathers = [
                pltpu.async_copy(h_hbm.at[sidx_v.at[blk0 + j]],
                                 rows_v.at[j], gsem)
                for j in range(K)
            ]
            for d in gathers:
                d.wait()
            scatters = [
                pltpu.async_copy(rows_v.at[j],
                                 acc_sh.at[didx_v.at[blk0 + j]],
                                 ssem, add=True)
                for j in range(K)
            ]
            for d in scatters:
                d.wait()

        plsc.subcore_barrier()
        pltpu.sync_copy(acc_sh.at[pl.ds(row0, ROWS_PER_TILE)],
                        out_hbm.at[c, pl.ds(row0, ROWS_PER_TILE)])

    return agg_kernel(h, src, dst, zeros64)


def _tc_prologue(x, y, f1w, f1b, f2w, f2b, c1w, degp):
    def body(x_ref, y_ref, f1w_ref, f1b_ref, f2w_ref, f2b_ref, c1w_ref,
             degp_ref, h1s_ref, dinv_ref):
        zx = jnp.dot(x_ref[...], f1w_ref[...],
                     preferred_element_type=jnp.float32) + f1b_ref[...]
        zy = jnp.dot(y_ref[...], f2w_ref[...],
                     preferred_element_type=jnp.float32) + f2b_ref[...]
        z = jnp.concatenate([zx, zy], axis=0)
        h1 = jnp.dot(z, c1w_ref[...], preferred_element_type=jnp.float32)
        deg = degp_ref[0, :N, :1] + degp_ref[1, :N, :1] + 1.0
        dinv = lax.rsqrt(deg)
        h1s_ref[...] = h1 * dinv
        dinv_ref[...] = dinv

    return pl.pallas_call(
        body,
        out_shape=(jax.ShapeDtypeStruct((N, D), jnp.float32),
                   jax.ShapeDtypeStruct((N, 1), jnp.float32)),
    )(x, y, f1w, f1b, f2w, f2b, c1w, degp)


def _tc_mid(p1, h1s, dinv, c1b, gamma, beta, c2w):
    def body(p_ref, h1s_ref, dinv_ref, c1b_ref, g_ref, b_ref, c2w_ref,
             h2s_ref):
        dinv = dinv_ref[...]
        t = (p_ref[0, :N] + p_ref[1, :N] + h1s_ref[...]) * dinv + c1b_ref[...]
        mean = jnp.mean(t, axis=0, keepdims=True)
        var = jnp.mean((t - mean) ** 2, axis=0, keepdims=True)
        r = (t - mean) * lax.rsqrt(var + BN_EPS) * g_ref[...] + b_ref[...]
        r = jnp.maximum(r, 0.0)
        h2 = jnp.dot(r, c2w_ref[...], preferred_element_type=jnp.float32)
        h2s_ref[...] = h2 * dinv

    return pl.pallas_call(
        body,
        out_shape=jax.ShapeDtypeStruct((N, D), jnp.float32),
    )(p1, h1s, dinv, c1b, gamma, beta, c2w)


def _tc_epilogue(p2, h2s, dinv, c2b):
    def body(p_ref, h2s_ref, dinv_ref, c2b_ref, out_ref):
        out_ref[...] = ((p_ref[0, :N] + p_ref[1, :N] + h2s_ref[...])
                        * dinv_ref[...] + c2b_ref[...])

    return pl.pallas_call(
        body,
        out_shape=jax.ShapeDtypeStruct((N, D), jnp.float32),
    )(p2, h2s, dinv, c2b)


def kernel(x, y, edge_index, fc1_W, fc1_b, fc2_W, fc2_b,
           conv1_W, conv1_b, conv2_W, conv2_b, bn_gamma, bn_beta):
    src = edge_index[0].reshape(NW, NBLK, B)
    dst = edge_index[1].reshape(NW, NBLK, B)
    zeros16 = jnp.zeros((N_PAD, DW), jnp.float32)
    zeros64 = jnp.zeros((N_PAD, D), jnp.float32)
    ones_blk = jnp.ones((B, DW), jnp.float32)

    degp = _sc_degree(dst, zeros16, ones_blk)
    h1s, dinv = _tc_prologue(x, y, fc1_W, fc1_b.reshape(1, -1),
                             fc2_W, fc2_b.reshape(1, -1), conv1_W, degp)
    p1 = _sc_aggregate(h1s, src, dst, zeros64)
    h2s = _tc_mid(p1, h1s, dinv, conv1_b.reshape(1, -1),
                  bn_gamma.reshape(1, -1), bn_beta.reshape(1, -1), conv2_W)
    p2 = _sc_aggregate(h2s, src, dst, zeros64)
    out = _tc_epilogue(p2, h2s, dinv, conv2_b.reshape(1, -1))
    return out
